# baseline (speedup 1.0000x reference)
import jax
import jax.numpy as jnp
from jax.experimental import pallas as pl
from jax.experimental.pallas import tpu as pltpu

_NUM_CLASSES = 81
_K = 12544
_KB = 896
_NK = _K // _KB
_NA = 512
_NB = 512
_N = _NA + _NB
_H = 1024
_RT = 128
_EPS = 1e-3
_PREC = jax.lax.Precision.HIGHEST


def _head_kernel(a_ref, b_ref, w1_ref, g1_ref, be1_ref,
                 w2_ref, g2_ref, be2_ref,
                 lw_ref, lb_ref, dw_ref, db_ref,
                 logits_ref, probs_ref, deltas_ref, acc_ref, x2_ref):
    k = pl.program_id(0)

    @pl.when(k == 0)
    def _():
        acc_ref[...] = jnp.zeros_like(acc_ref)

    w = w1_ref[...]
    acc_ref[0:_NA, :] += jnp.dot(a_ref[...], w, precision=_PREC,
                                 preferred_element_type=jnp.float32)
    acc_ref[_NA:_N, :] += jnp.dot(b_ref[...], w, precision=_PREC,
                                  preferred_element_type=jnp.float32)

    @pl.when(k == _NK - 1)
    def _():
        mean1 = jnp.mean(acc_ref[...], axis=0, keepdims=True)
        var1 = jnp.mean((acc_ref[...] - mean1) ** 2, axis=0, keepdims=True)
        scale1 = g1_ref[...] * jax.lax.rsqrt(var1 + _EPS)
        shift1 = be1_ref[...] - mean1 * scale1

        def body1(i, carry):
            r = i * _RT
            xt = jnp.maximum(acc_ref[pl.ds(r, _RT), :] * scale1 + shift1, 0.0)
            x2_ref[pl.ds(r, _RT), :] = jnp.dot(
                xt, w2_ref[...], precision=_PREC,
                preferred_element_type=jnp.float32)
            return carry

        jax.lax.fori_loop(0, _N // _RT, body1, 0, unroll=False)

        mean2 = jnp.mean(x2_ref[...], axis=0, keepdims=True)
        var2 = jnp.mean((x2_ref[...] - mean2) ** 2, axis=0, keepdims=True)
        scale2 = g2_ref[...] * jax.lax.rsqrt(var2 + _EPS)
        shift2 = be2_ref[...] - mean2 * scale2

        def body2(i, carry):
            r = i * _RT
            xt = jnp.maximum(x2_ref[pl.ds(r, _RT), :] * scale2 + shift2, 0.0)
            logits = jnp.dot(xt, lw_ref[...], precision=_PREC,
                             preferred_element_type=jnp.float32) + lb_ref[...]
            logits_ref[pl.ds(r, _RT), :] = logits
            m = jnp.max(logits, axis=-1, keepdims=True)
            e = jnp.exp(logits - m)
            probs_ref[pl.ds(r, _RT), :] = e / jnp.sum(e, axis=-1, keepdims=True)
            deltas_ref[pl.ds(r, _RT), :] = jnp.dot(
                xt, dw_ref[...], precision=_PREC,
                preferred_element_type=jnp.float32) + db_ref[...]
            return carry

        jax.lax.fori_loop(0, _N // _RT, body2, 0, unroll=False)


def kernel(pooled_rois_a, pooled_rois_b, conv1_w, conv1_b, bn1_gamma, bn1_beta,
           conv2_w, conv2_b, bn2_gamma, bn2_beta, logits_w, logits_b,
           delta_w, delta_b):
    del conv1_b, conv2_b
    a2 = pooled_rois_a.reshape(_NA, _K)
    b2 = pooled_rois_b.reshape(_NB, _K)
    row = lambda v: v.reshape(1, -1)
    full = lambda shape: pl.BlockSpec(shape, lambda k: (0, 0))

    logits, probs, deltas = pl.pallas_call(
        _head_kernel,
        grid=(_NK,),
        in_specs=[
            pl.BlockSpec((_NA, _KB), lambda k: (0, k)),
            pl.BlockSpec((_NB, _KB), lambda k: (0, k)),
            pl.BlockSpec((_KB, _H), lambda k: (k, 0)),
            full((1, _H)), full((1, _H)),
            full((_H, _H)), full((1, _H)), full((1, _H)),
            full((_H, _NUM_CLASSES)), full((1, _NUM_CLASSES)),
            full((_H, 4 * _NUM_CLASSES)), full((1, 4 * _NUM_CLASSES)),
        ],
        out_specs=[
            full((_N, _NUM_CLASSES)),
            full((_N, _NUM_CLASSES)),
            full((_N, 4 * _NUM_CLASSES)),
        ],
        out_shape=[
            jax.ShapeDtypeStruct((_N, _NUM_CLASSES), jnp.float32),
            jax.ShapeDtypeStruct((_N, _NUM_CLASSES), jnp.float32),
            jax.ShapeDtypeStruct((_N, 4 * _NUM_CLASSES), jnp.float32),
        ],
        scratch_shapes=[
            pltpu.VMEM((_N, _H), jnp.float32),
            pltpu.VMEM((_N, _H), jnp.float32),
        ],
        compiler_params=pltpu.CompilerParams(
            dimension_semantics=("arbitrary",)),
    )(a2, b2, conv1_w,
      row(bn1_gamma), row(bn1_beta),
      conv2_w, row(bn2_gamma), row(bn2_beta),
      logits_w, row(logits_b), delta_w, row(delta_b))

    return (logits, probs, deltas.reshape(_N, _NUM_CLASSES, 4))


# speedup vs baseline: 2.0501x; 2.0501x over previous
import jax
import jax.numpy as jnp
from jax.experimental import pallas as pl
from jax.experimental.pallas import tpu as pltpu

_NUM_CLASSES = 81
_K = 12544
_KB = 896
_NK = _K // _KB
_NA = 512
_NB = 512
_N = _NA + _NB
_H = 1024
_RT = 128
_EPS = 1e-3
_PREC = jax.lax.Precision.DEFAULT


def _head_kernel(a_ref, b_ref, w1_ref, g1_ref, be1_ref,
                 w2_ref, g2_ref, be2_ref,
                 lw_ref, lb_ref, dw_ref, db_ref,
                 logits_ref, probs_ref, deltas_ref, acc_ref, x2_ref):
    k = pl.program_id(0)

    @pl.when(k == 0)
    def _():
        acc_ref[...] = jnp.zeros_like(acc_ref)

    w = w1_ref[...]
    acc_ref[0:_NA, :] += jnp.dot(a_ref[...], w, precision=_PREC,
                                 preferred_element_type=jnp.float32)
    acc_ref[_NA:_N, :] += jnp.dot(b_ref[...], w, precision=_PREC,
                                  preferred_element_type=jnp.float32)

    @pl.when(k == _NK - 1)
    def _():
        mean1 = jnp.mean(acc_ref[...], axis=0, keepdims=True)
        var1 = jnp.mean((acc_ref[...] - mean1) ** 2, axis=0, keepdims=True)
        scale1 = g1_ref[...] * jax.lax.rsqrt(var1 + _EPS)
        shift1 = be1_ref[...] - mean1 * scale1

        def body1(i, carry):
            r = i * _RT
            xt = jnp.maximum(acc_ref[pl.ds(r, _RT), :] * scale1 + shift1, 0.0)
            x2_ref[pl.ds(r, _RT), :] = jnp.dot(
                xt, w2_ref[...], precision=_PREC,
                preferred_element_type=jnp.float32)
            return carry

        jax.lax.fori_loop(0, _N // _RT, body1, 0, unroll=False)

        mean2 = jnp.mean(x2_ref[...], axis=0, keepdims=True)
        var2 = jnp.mean((x2_ref[...] - mean2) ** 2, axis=0, keepdims=True)
        scale2 = g2_ref[...] * jax.lax.rsqrt(var2 + _EPS)
        shift2 = be2_ref[...] - mean2 * scale2

        def body2(i, carry):
            r = i * _RT
            xt = jnp.maximum(x2_ref[pl.ds(r, _RT), :] * scale2 + shift2, 0.0)
            logits = jnp.dot(xt, lw_ref[...], precision=_PREC,
                             preferred_element_type=jnp.float32) + lb_ref[...]
            logits_ref[pl.ds(r, _RT), :] = logits
            m = jnp.max(logits, axis=-1, keepdims=True)
            e = jnp.exp(logits - m)
            probs_ref[pl.ds(r, _RT), :] = e / jnp.sum(e, axis=-1, keepdims=True)
            deltas_ref[pl.ds(r, _RT), :] = jnp.dot(
                xt, dw_ref[...], precision=_PREC,
                preferred_element_type=jnp.float32) + db_ref[...]
            return carry

        jax.lax.fori_loop(0, _N // _RT, body2, 0, unroll=False)


def kernel(pooled_rois_a, pooled_rois_b, conv1_w, conv1_b, bn1_gamma, bn1_beta,
           conv2_w, conv2_b, bn2_gamma, bn2_beta, logits_w, logits_b,
           delta_w, delta_b):
    del conv1_b, conv2_b
    a2 = pooled_rois_a.reshape(_NA, _K)
    b2 = pooled_rois_b.reshape(_NB, _K)
    row = lambda v: v.reshape(1, -1)
    full = lambda shape: pl.BlockSpec(shape, lambda k: (0, 0))

    logits, probs, deltas = pl.pallas_call(
        _head_kernel,
        grid=(_NK,),
        in_specs=[
            pl.BlockSpec((_NA, _KB), lambda k: (0, k)),
            pl.BlockSpec((_NB, _KB), lambda k: (0, k)),
            pl.BlockSpec((_KB, _H), lambda k: (k, 0)),
            full((1, _H)), full((1, _H)),
            full((_H, _H)), full((1, _H)), full((1, _H)),
            full((_H, _NUM_CLASSES)), full((1, _NUM_CLASSES)),
            full((_H, 4 * _NUM_CLASSES)), full((1, 4 * _NUM_CLASSES)),
        ],
        out_specs=[
            full((_N, _NUM_CLASSES)),
            full((_N, _NUM_CLASSES)),
            full((_N, 4 * _NUM_CLASSES)),
        ],
        out_shape=[
            jax.ShapeDtypeStruct((_N, _NUM_CLASSES), jnp.float32),
            jax.ShapeDtypeStruct((_N, _NUM_CLASSES), jnp.float32),
            jax.ShapeDtypeStruct((_N, 4 * _NUM_CLASSES), jnp.float32),
        ],
        scratch_shapes=[
            pltpu.VMEM((_N, _H), jnp.float32),
            pltpu.VMEM((_N, _H), jnp.float32),
        ],
        compiler_params=pltpu.CompilerParams(
            dimension_semantics=("arbitrary",)),
    )(a2, b2, conv1_w,
      row(bn1_gamma), row(bn1_beta),
      conv2_w, row(bn2_gamma), row(bn2_beta),
      logits_w, row(logits_b), delta_w, row(delta_b))

    return (logits, probs, deltas.reshape(_N, _NUM_CLASSES, 4))


# speedup vs baseline: 2.0969x; 1.0229x over previous
import jax
import jax.numpy as jnp
from jax.experimental import pallas as pl
from jax.experimental.pallas import tpu as pltpu

_NUM_CLASSES = 81
_K = 12544
_KB = 1792
_NK = _K // _KB
_NA = 512
_NB = 512
_N = _NA + _NB
_H = 1024
_RT = 128
_EPS = 1e-3


def _dot(a, b):
    return jnp.dot(a, b, preferred_element_type=jnp.float32)


def _head_kernel(a_ref, b_ref, w1_ref, g1_ref, be1_ref,
                 w2_ref, g2_ref, be2_ref,
                 lw_ref, lb_ref, dw_ref, db_ref,
                 logits_ref, probs_ref, deltas_ref, acc_ref, x2_ref):
    k = pl.program_id(0)
    w = w1_ref[...]

    @pl.when(k == 0)
    def _():
        acc_ref[0:_NA, :] = _dot(a_ref[...], w)
        acc_ref[_NA:_N, :] = _dot(b_ref[...], w)

    @pl.when(k > 0)
    def _():
        acc_ref[0:_NA, :] += _dot(a_ref[...], w)
        acc_ref[_NA:_N, :] += _dot(b_ref[...], w)

    @pl.when(k == _NK - 1)
    def _():
        zero = jnp.zeros((1, _H), jnp.float32)

        def stats1(i, carry):
            s, ss = carry
            t = acc_ref[pl.ds(i * _RT, _RT), :]
            return (s + jnp.sum(t, axis=0, keepdims=True),
                    ss + jnp.sum(t * t, axis=0, keepdims=True))

        s1, ss1 = jax.lax.fori_loop(0, _N // _RT, stats1, (zero, zero))
        mean1 = s1 * (1.0 / _N)
        var1 = ss1 * (1.0 / _N) - mean1 * mean1
        scale1 = g1_ref[...] * jax.lax.rsqrt(var1 + _EPS)
        shift1 = be1_ref[...] - mean1 * scale1

        def body1(i, carry):
            s, ss = carry
            r = i * _RT
            xt = jnp.maximum(acc_ref[pl.ds(r, _RT), :] * scale1 + shift1, 0.0)
            x2t = _dot(xt, w2_ref[...])
            x2_ref[pl.ds(r, _RT), :] = x2t
            return (s + jnp.sum(x2t, axis=0, keepdims=True),
                    ss + jnp.sum(x2t * x2t, axis=0, keepdims=True))

        s2, ss2 = jax.lax.fori_loop(0, _N // _RT, body1, (zero, zero))
        mean2 = s2 * (1.0 / _N)
        var2 = ss2 * (1.0 / _N) - mean2 * mean2
        scale2 = g2_ref[...] * jax.lax.rsqrt(var2 + _EPS)
        shift2 = be2_ref[...] - mean2 * scale2

        def body2(i, carry):
            r = i * _RT
            xt = jnp.maximum(x2_ref[pl.ds(r, _RT), :] * scale2 + shift2, 0.0)
            logits = _dot(xt, lw_ref[...]) + lb_ref[...]
            logits_ref[pl.ds(r, _RT), :] = logits
            m = jnp.max(logits, axis=-1, keepdims=True)
            e = jnp.exp(logits - m)
            probs_ref[pl.ds(r, _RT), :] = e / jnp.sum(e, axis=-1, keepdims=True)
            deltas_ref[pl.ds(r, _RT), :] = _dot(xt, dw_ref[...]) + db_ref[...]
            return carry

        jax.lax.fori_loop(0, _N // _RT, body2, 0)


def kernel(pooled_rois_a, pooled_rois_b, conv1_w, conv1_b, bn1_gamma, bn1_beta,
           conv2_w, conv2_b, bn2_gamma, bn2_beta, logits_w, logits_b,
           delta_w, delta_b):
    del conv1_b, conv2_b
    a2 = pooled_rois_a.reshape(_NA, _K)
    b2 = pooled_rois_b.reshape(_NB, _K)
    row = lambda v: v.reshape(1, -1)
    full = lambda shape: pl.BlockSpec(shape, lambda k: (0, 0))

    logits, probs, deltas = pl.pallas_call(
        _head_kernel,
        grid=(_NK,),
        in_specs=[
            pl.BlockSpec((_NA, _KB), lambda k: (0, k)),
            pl.BlockSpec((_NB, _KB), lambda k: (0, k)),
            pl.BlockSpec((_KB, _H), lambda k: (k, 0)),
            full((1, _H)), full((1, _H)),
            full((_H, _H)), full((1, _H)), full((1, _H)),
            full((_H, _NUM_CLASSES)), full((1, _NUM_CLASSES)),
            full((_H, 4 * _NUM_CLASSES)), full((1, 4 * _NUM_CLASSES)),
        ],
        out_specs=[
            full((_N, _NUM_CLASSES)),
            full((_N, _NUM_CLASSES)),
            full((_N, 4 * _NUM_CLASSES)),
        ],
        out_shape=[
            jax.ShapeDtypeStruct((_N, _NUM_CLASSES), jnp.float32),
            jax.ShapeDtypeStruct((_N, _NUM_CLASSES), jnp.float32),
            jax.ShapeDtypeStruct((_N, 4 * _NUM_CLASSES), jnp.float32),
        ],
        scratch_shapes=[
            pltpu.VMEM((_N, _H), jnp.float32),
            pltpu.VMEM((_N, _H), jnp.float32),
        ],
        compiler_params=pltpu.CompilerParams(
            dimension_semantics=("arbitrary",)),
    )(a2, b2, conv1_w,
      row(bn1_gamma), row(bn1_beta),
      conv2_w, row(bn2_gamma), row(bn2_beta),
      logits_w, row(logits_b), delta_w, row(delta_b))

    return (logits, probs, deltas.reshape(_N, _NUM_CLASSES, 4))
